# baseline (speedup 1.0000x reference)
import functools

import numpy as np
import jax
import jax.numpy as jnp
from jax import lax
from jax.experimental import pallas as pl
from jax.experimental.pallas import tpu as pltpu
from jax.experimental.pallas import tpu_sc as plsc

_N, _E, _D, _H = 10000, 320000, 128, 256
_NP = 10240
_EP = 327680
_ER = _EP // 128
_DUMMY = 10100
_ROWS_PER_TILE = _NP // 16
_CCAP = 4096
_CER = 32 * _CCAP // 128

_TIMESTEP, _START_T = 10000, 9000
_betas = np.linspace(1e-4, 0.02, _TIMESTEP, dtype=np.float64)
_ac = np.cumprod(1.0 - _betas)
_SQRT_AC = np.sqrt(_ac).astype(np.float32)
_SQRT_1MAC = np.sqrt(1.0 - _ac).astype(np.float32)


def _np(a):
    return np.asarray(jax.device_get(a))


@functools.lru_cache(maxsize=1)
def _mask_consts():
    with jax.default_device(jax.local_devices(backend="cpu")[0]):
        return _mask_consts_impl()


def _mask_consts_impl():
    mkey = jax.random.key(42)
    k1, k2, k3, k4, k5, k6 = jax.random.split(mkey, 6)
    n = _N
    num_mask = int(0.3 * n)
    num_noise = int(0.1 * num_mask)
    perm = _np(jax.random.permutation(k1, n))
    mask_nodes = perm[:num_mask]
    perm_mask = _np(jax.random.permutation(k2, num_mask))
    token_nodes = mask_nodes[perm_mask[: int(0.9 * num_mask)]]
    noise_nodes = mask_nodes[perm_mask[num_mask - num_noise:]]
    noise_chosen = _np(jax.random.permutation(k3, n))[:num_noise]
    t = _np(jax.random.randint(k4, (num_mask,), _START_T, _TIMESTEP))
    noise = _np(jax.random.normal(k5, (num_mask, _H), dtype=jnp.float32))
    perm_idx = _np(jax.random.permutation(k6, num_mask))
    remask_nodes = mask_nodes[perm_idx[: int(0.6 * num_mask)]]

    tf = t.astype(np.float32)
    a_c = _SQRT_AC[t] / tf
    b_c = _SQRT_1MAC[t] / tf

    g = np.arange(_NP, dtype=np.int32)
    g[token_nodes] = _N
    g[noise_nodes] = noise_chosen

    ca = np.ones((_NP, 1), np.float32)
    ca[mask_nodes, 0] = a_c
    ca[remask_nodes, 0] = 0.0
    cb = np.zeros((_NP, _H), np.float32)
    cb[mask_nodes] = b_c[:, None] * noise
    cb[remask_nodes] = 0.0
    rm = np.zeros((_NP, 1), np.float32)
    rm[remask_nodes, 0] = 1.0
    wm = np.zeros((_NP, 1), np.float32)
    wm[mask_nodes, 0] = 1.0
    flag = np.zeros((_NP,), np.int32)
    flag[mask_nodes] = 1
    return dict(g=g, ca=ca, cb=cb, rm=rm, wm=wm, flag=flag)


def _mesh():
    return plsc.VectorSubcoreMesh(
        core_axis_name="c", subcore_axis_name="s", num_cores=2,
        num_subcores=16)


@functools.lru_cache(maxsize=1)
def _make_prelude():
    nchunk = _NP // 128
    erpt = _ER // 32
    crpt = _CCAP // 128

    def body(tab, g2d, src2d, dst2d, flag_h, fsrc_h, fdst_h,
             outx, csrc, cdst,
             idxv, rowsb, flagv, srcblk, dstblk, csb, cdb, sem):
        wid = lax.axis_index("c") * 16 + lax.axis_index("s")
        for b in range(3):
            cid = wid + b * 32

            @pl.when(cid < nchunk)
            def _():
                pltpu.sync_copy(g2d.at[pl.ds(cid, 1)], idxv)
                pltpu.async_copy(tab.at[idxv.at[0]], rowsb, sem).wait()
                pltpu.sync_copy(rowsb, outx.at[pl.ds(cid * 128, 128)])

        pltpu.sync_copy(flag_h, flagv)
        pltpu.sync_copy(src2d.at[pl.ds(wid * erpt, erpt)], srcblk)
        pltpu.sync_copy(dst2d.at[pl.ds(wid * erpt, erpt)], dstblk)
        pltpu.sync_copy(fsrc_h, csb)
        pltpu.sync_copy(fdst_h, cdb)

        def row(r, off):
            for v in range(8):
                dv = dstblk[r, pl.ds(v * 16, 16)]
                sv = srcblk[r, pl.ds(v * 16, 16)]
                fl = plsc.load_gather(flagv, [dv])
                m = fl > 0
                o = jnp.minimum(off, _CCAP - 16)
                plsc.store_compressed(csb.at[pl.ds(o, 16)], sv, mask=m)
                plsc.store_compressed(cdb.at[pl.ds(o, 16)], dv, mask=m)
                cnt = plsc.all_reduce_population_count(m)
                off = off + cnt[0]
            return off

        lax.fori_loop(0, erpt, row, jnp.int32(0))
        pltpu.sync_copy(csb, csrc.at[wid])
        pltpu.sync_copy(cdb, cdst.at[wid])

    return pl.kernel(
        body,
        out_type=(jax.ShapeDtypeStruct((_NP, 128), jnp.float32),
                  jax.ShapeDtypeStruct((32, _CCAP), jnp.int32),
                  jax.ShapeDtypeStruct((32, _CCAP), jnp.int32)),
        mesh=_mesh(),
        compiler_params=pltpu.CompilerParams(needs_layout_passes=False),
        scratch_types=(
            pltpu.VMEM((1, 128), jnp.int32),
            pltpu.VMEM((128, 128), jnp.float32),
            pltpu.VMEM((_NP,), jnp.int32),
            pltpu.VMEM((erpt, 128), jnp.int32),
            pltpu.VMEM((erpt, 128), jnp.int32),
            pltpu.VMEM((_CCAP,), jnp.int32),
            pltpu.VMEM((_CCAP,), jnp.int32),
            pltpu.SemaphoreType.DMA,
        ))


@functools.lru_cache(maxsize=4)
def _make_segsum(split_by_core: bool, with_deg: bool, er: int):
    outs = [jax.ShapeDtypeStruct((2, _NP, 128), jnp.float32)]
    if with_deg:
        outs.append(jax.ShapeDtypeStruct((32, _NP), jnp.float32))
    scratch = [
        pltpu.VMEM((16, 128), jnp.int32),
        pltpu.VMEM((16, 128), jnp.int32),
        pltpu.VMEM((2 * 128, 128), jnp.float32),
        pltpu.VMEM_SHARED((_NP, 128), jnp.float32),
        pltpu.SemaphoreType.DMA,
        pltpu.SemaphoreType.DMA,
        pltpu.SemaphoreType.DMA,
        pltpu.SemaphoreType.DMA,
        pltpu.SemaphoreType.DMA,
        pltpu.SemaphoreType.DMA,
    ]
    if not split_by_core:
        scratch.append(pltpu.VMEM((16, 128), jnp.int32))
    if with_deg:
        scratch.append(pltpu.VMEM((_NP,), jnp.float32))

    def body(*refs):
        idxv = None
        if with_deg:
            (tab, src2d, dst2d, zrows, zvec,
             out, outdeg, srcv, dstv, rows, acc,
             sg0, sg1, ss0, ss1, sb0, sb1, degpart) = refs
        else:
            (tab, src2d, dst2d, zrows,
             out, srcv, dstv, rows, acc,
             sg0, sg1, ss0, ss1, sb0, sb1, idxv) = refs
        semg = (sg0, sg1)
        sems = (ss0, ss1)
        semb = (sb0, sb1)
        c = lax.axis_index("c")
        s = lax.axis_index("s")
        r0 = s * _ROWS_PER_TILE
        pltpu.sync_copy(zrows.at[pl.ds(r0, _ROWS_PER_TILE)],
                        acc.at[pl.ds(r0, _ROWS_PER_TILE)])
        if with_deg:
            pltpu.sync_copy(zvec, degpart)
        plsc.subcore_barrier()

        if split_by_core:
            wid = c * 16 + s
            nstep = er // 32
            rb0 = wid * nstep
        else:
            nstep = er // 16
            rb0 = s * nstep
        coff = c * _NP
        nsb = nstep // 8

        ones16 = jnp.full((16,), 1.0, jnp.float32)

        def sblock_descs(sb, sbp):
            return (
                pltpu.make_async_copy(src2d.at[pl.ds(rb0 + sb * 8, 8)],
                                      srcv.at[pl.ds(sbp * 8, 8)], semb[sbp]),
                pltpu.make_async_copy(dst2d.at[pl.ds(rb0 + sb * 8, 8)],
                                      dstv.at[pl.ds(sbp * 8, 8)], semb[sbp]),
            )

        def gather_desc(row, rp):
            idx = srcv if split_by_core else idxv
            return pltpu.make_async_copy(
                tab.at[idx.at[row]], rows.at[pl.ds(rp * 128, 128)],
                semg[rp])

        def scatter_desc(row, rp):
            return pltpu.make_async_copy(
                rows.at[pl.ds(rp * 128, 128)], acc.at[dstv.at[row]],
                sems[rp])

        for d in sblock_descs(0, 0):
            d.start()

        def pairblock(i, carry):
            for sbp in (0, 1):
                sb = 2 * i + sbp
                for s8 in range(8):
                    step = sb * 8 + s8
                    rp = s8 % 2
                    row = sbp * 8 + s8

                    @pl.when(step >= 1)
                    def _():
                        scatter_desc(row, 1 - rp).wait()

                    if s8 == 0:
                        @pl.when(sb + 1 < nsb)
                        def _():
                            for d in sblock_descs(sb + 1, 1 - sbp):
                                d.start()

                        for d in sblock_descs(sb, sbp):
                            d.wait()
                        if not split_by_core:
                            for rr in range(8):
                                for v in range(8):
                                    sv = srcv[sbp * 8 + rr,
                                              pl.ds(v * 16, 16)]
                                    idxv[sbp * 8 + rr,
                                         pl.ds(v * 16, 16)] = sv + coff
                        gather_desc(row, 0).start()
                        gather_desc(row + 1, 1).start()
                    elif s8 < 7:
                        gather_desc(row + 1, 1 - rp).start()

                    gather_desc(row, rp).wait()
                    pltpu.async_copy(rows.at[pl.ds(rp * 128, 128)],
                                     acc.at[dstv.at[row]], sems[rp],
                                     add=True)
                    if with_deg:
                        for v in range(8):
                            dv = dstv[row, pl.ds(v * 16, 16)]
                            plsc.addupdate_scatter(degpart, [dv], ones16)
            return carry

        lax.fori_loop(0, nsb // 2, pairblock, 0)
        scatter_desc(15, 1).wait()
        plsc.subcore_barrier()
        pltpu.sync_copy(acc.at[pl.ds(r0, _ROWS_PER_TILE)],
                        out.at[c, pl.ds(r0, _ROWS_PER_TILE)])
        if with_deg:
            pltpu.sync_copy(degpart, outdeg.at[c * 16 + s])

    out_type = tuple(outs) if len(outs) > 1 else outs[0]
    return pl.kernel(
        body, out_type=out_type, mesh=_mesh(),
        compiler_params=pltpu.CompilerParams(needs_layout_passes=False),
        scratch_types=tuple(scratch))


_BLK = 1024
_GRID = _NP // _BLK


def _vec_spec():
    return pl.BlockSpec((_BLK, 1), lambda i: (i, 0))


def _mat_spec():
    return pl.BlockSpec((_BLK, 128), lambda i: (i, 0))


def _deg_spec():
    return pl.BlockSpec((32, _BLK), lambda i: (0, i))


def _deg_of(dr):
    return jnp.maximum(jnp.sum(dr[...], axis=0), 1.0)[:, None]


def _tc_encode1(p0, p1, dg, w1):
    def body(p0r, p1r, dgr, w1r, outr):
        deg = _deg_of(dgr)
        agg = (p0r[...] + p1r[...]) / deg
        h = jnp.dot(agg, w1r[...], preferred_element_type=jnp.float32)
        h = jnp.maximum(h, 0.0)
        outr[0] = h[:, :128]
        outr[1] = h[:, 128:]

    return pl.pallas_call(
        body,
        grid=(_GRID,),
        in_specs=[_mat_spec(), _mat_spec(), _deg_spec(),
                  pl.BlockSpec((128, _H), lambda i: (0, 0))],
        out_specs=pl.BlockSpec((2, _BLK, 128), lambda i: (0, i, 0)),
        out_shape=jax.ShapeDtypeStruct((2, _NP, 128), jnp.float32),
    )(p0, p1, dg, w1)


def _tc_encode2(alo, ahi, dg, w2, we2d, ca, cb, rm, retok):
    def body(alor, ahir, dgr, w2r, wer, car, cbr, rmr, rtr, outr):
        deg = _deg_of(dgr)
        w2 = w2r[...]
        enc = (jnp.dot(alor[...] / deg, w2[:128],
                       preferred_element_type=jnp.float32) +
               jnp.dot(ahir[...] / deg, w2[128:],
                       preferred_element_type=jnp.float32))
        enc = jnp.maximum(enc, 0.0)
        rep = jnp.dot(enc, wer[...], preferred_element_type=jnp.float32)
        rep = car[...] * rep + cbr[...] + rmr[...] * rtr[...][0]
        outr[0] = rep[:, :128]
        outr[1] = rep[:, 128:]

    return pl.pallas_call(
        body,
        grid=(_GRID,),
        in_specs=[_mat_spec(), _mat_spec(), _deg_spec(),
                  pl.BlockSpec((_H, _H), lambda i: (0, 0)),
                  pl.BlockSpec((_H, _H), lambda i: (0, 0)),
                  _vec_spec(),
                  pl.BlockSpec((_BLK, _H), lambda i: (i, 0)),
                  _vec_spec(),
                  pl.BlockSpec((8, _H), lambda i: (0, 0))],
        out_specs=pl.BlockSpec((2, _BLK, 128), lambda i: (0, i, 0)),
        out_shape=jax.ShapeDtypeStruct((2, _NP, 128), jnp.float32),
    )(alo, ahi, dg, w2, we2d, ca, cb, rm, retok)


def _tc_decode_loss(alo, ahi, dg, xp, wd, wm):
    def body(alor, ahir, dgr, xr, wdr, wmr, outr):
        i = pl.program_id(0)
        deg = _deg_of(dgr)
        wd = wdr[...]
        y = (jnp.dot(alor[...] / deg, wd[:128],
                     preferred_element_type=jnp.float32) +
             jnp.dot(ahir[...] / deg, wd[128:],
                     preferred_element_type=jnp.float32))
        x = xr[...]
        xn = x / (jnp.sqrt(jnp.sum(x * x, axis=-1, keepdims=True)) + 1e-8)
        yn = y / (jnp.sqrt(jnp.sum(y * y, axis=-1, keepdims=True)) + 1e-8)
        cos = jnp.sum(xn * yn, axis=-1, keepdims=True)
        li = (1.0 - cos) ** 2 * wmr[...]
        part = jnp.sum(li) * (1.0 / 3000.0)

        @pl.when(i == 0)
        def _():
            outr[...] = jnp.zeros_like(outr)

        outr[...] += part

    return pl.pallas_call(
        body,
        grid=(_GRID,),
        in_specs=[_mat_spec(), _mat_spec(), _deg_spec(),
                  _mat_spec(),
                  pl.BlockSpec((_H, 128), lambda i: (0, 0)),
                  _vec_spec()],
        out_specs=pl.BlockSpec((8, 128), lambda i: (0, 0)),
        out_shape=jax.ShapeDtypeStruct((8, 128), jnp.float32),
    )(alo, ahi, dg, xp, wd, wm)


_CS = _mask_consts()


def kernel(x, edge_index, epoch, W1, W2, enc_mask_token, W_e2d,
           re_enc_mask_token, Wd):
    cs = _CS
    f32 = jnp.float32

    tab1 = jnp.concatenate(
        [x, enc_mask_token,
         jnp.zeros((_NP - _N - 1, _D), f32)], axis=0)
    xp = jnp.concatenate([x, jnp.zeros((_NP - _N, _D), f32)], axis=0)

    src = jnp.concatenate(
        [edge_index[0], jnp.zeros((_EP - _E,), jnp.int32)]).reshape(_ER, 128)
    dst = jnp.concatenate(
        [edge_index[1],
         jnp.full((_EP - _E,), _DUMMY, jnp.int32)]).reshape(_ER, 128)

    zrows = jnp.zeros((_NP, 128), f32)
    zvec = jnp.zeros((_NP,), f32)
    g2d = jnp.asarray(cs["g"]).reshape(_NP // 128, 128)
    flag = jnp.asarray(cs["flag"])
    fsrc = jnp.zeros((_CCAP,), jnp.int32)
    fdst = jnp.full((_CCAP,), _DUMMY, jnp.int32)

    outx, csrc, cdst = _make_prelude()(tab1, g2d, src, dst, flag, fsrc, fdst)
    csrc = csrc.reshape(_CER, 128)
    cdst = cdst.reshape(_CER, 128)

    seg1 = _make_segsum(True, True, _ER)
    p, dg = seg1(outx, src, dst, zrows, zvec)

    h1 = _tc_encode1(p[0], p[1], dg, W1)

    seg2 = _make_segsum(False, False, _ER)
    a2 = seg2(h1.reshape(2 * _NP, 128), src, dst, zrows)

    retok = jnp.broadcast_to(re_enc_mask_token, (8, _H))
    rep = _tc_encode2(a2[0], a2[1], dg, W2, W_e2d,
                      jnp.asarray(cs["ca"]), jnp.asarray(cs["cb"]),
                      jnp.asarray(cs["rm"]), retok)

    seg3 = _make_segsum(False, False, _CER)
    a3 = seg3(rep.reshape(2 * _NP, 128), csrc, cdst, zrows)

    out = _tc_decode_loss(a3[0], a3[1], dg, xp, Wd,
                          jnp.asarray(cs["wm"]))
    return out[0, 0]


# speedup vs baseline: 1.0006x; 1.0006x over previous
import functools

import numpy as np
import jax
import jax.numpy as jnp
from jax import lax
from jax.experimental import pallas as pl
from jax.experimental.pallas import tpu as pltpu
from jax.experimental.pallas import tpu_sc as plsc

_N, _E, _D, _H = 10000, 320000, 128, 256
_NP = 10240
_EP = 327680
_ER = _EP // 128
_DUMMY = 10100
_ROWS_PER_TILE = _NP // 16
_CCAP = 4096
_CER = 32 * _CCAP // 128

_TIMESTEP, _START_T = 10000, 9000
_betas = np.linspace(1e-4, 0.02, _TIMESTEP, dtype=np.float64)
_ac = np.cumprod(1.0 - _betas)
_SQRT_AC = np.sqrt(_ac).astype(np.float32)
_SQRT_1MAC = np.sqrt(1.0 - _ac).astype(np.float32)


def _np(a):
    return np.asarray(jax.device_get(a))


@functools.lru_cache(maxsize=1)
def _mask_consts():
    with jax.default_device(jax.local_devices(backend="cpu")[0]):
        return _mask_consts_impl()


def _mask_consts_impl():
    mkey = jax.random.key(42)
    k1, k2, k3, k4, k5, k6 = jax.random.split(mkey, 6)
    n = _N
    num_mask = int(0.3 * n)
    num_noise = int(0.1 * num_mask)
    perm = _np(jax.random.permutation(k1, n))
    mask_nodes = perm[:num_mask]
    perm_mask = _np(jax.random.permutation(k2, num_mask))
    token_nodes = mask_nodes[perm_mask[: int(0.9 * num_mask)]]
    noise_nodes = mask_nodes[perm_mask[num_mask - num_noise:]]
    noise_chosen = _np(jax.random.permutation(k3, n))[:num_noise]
    t = _np(jax.random.randint(k4, (num_mask,), _START_T, _TIMESTEP))
    noise = _np(jax.random.normal(k5, (num_mask, _H), dtype=jnp.float32))
    perm_idx = _np(jax.random.permutation(k6, num_mask))
    remask_nodes = mask_nodes[perm_idx[: int(0.6 * num_mask)]]

    tf = t.astype(np.float32)
    a_c = _SQRT_AC[t] / tf
    b_c = _SQRT_1MAC[t] / tf

    g = np.arange(_NP, dtype=np.int32)
    g[token_nodes] = _N
    g[noise_nodes] = noise_chosen

    ca = np.ones((_NP, 1), np.float32)
    ca[mask_nodes, 0] = a_c
    ca[remask_nodes, 0] = 0.0
    cb = np.zeros((_NP, _H), np.float32)
    cb[mask_nodes] = b_c[:, None] * noise
    cb[remask_nodes] = 0.0
    rm = np.zeros((_NP, 1), np.float32)
    rm[remask_nodes, 0] = 1.0
    wm = np.zeros((_NP, 1), np.float32)
    wm[mask_nodes, 0] = 1.0
    flag = np.zeros((_NP,), np.int32)
    flag[mask_nodes] = 1
    return dict(g=g, ca=ca, cb=cb, rm=rm, wm=wm, flag=flag)


def _mesh():
    return plsc.VectorSubcoreMesh(
        core_axis_name="c", subcore_axis_name="s", num_cores=2,
        num_subcores=16)


@functools.lru_cache(maxsize=1)
def _make_prelude():
    nchunk = _NP // 128
    erpt = _ER // 32
    crpt = _CCAP // 128

    def body(tab, g2d, src2d, dst2d, flag_h, fsrc_h, fdst_h,
             outx, csrc, cdst,
             idxv, rowsb, flagv, srcblk, dstblk, csb, cdb, sem):
        wid = lax.axis_index("c") * 16 + lax.axis_index("s")
        for b in range(3):
            cid = wid + b * 32

            @pl.when(cid < nchunk)
            def _():
                pltpu.sync_copy(g2d.at[pl.ds(cid, 1)], idxv)
                pltpu.async_copy(tab.at[idxv.at[0]], rowsb, sem).wait()
                pltpu.sync_copy(rowsb, outx.at[pl.ds(cid * 128, 128)])

        pltpu.sync_copy(flag_h, flagv)
        pltpu.sync_copy(src2d.at[pl.ds(wid * erpt, erpt)], srcblk)
        pltpu.sync_copy(dst2d.at[pl.ds(wid * erpt, erpt)], dstblk)
        pltpu.sync_copy(fsrc_h, csb)
        pltpu.sync_copy(fdst_h, cdb)

        def row(r, off):
            for v in range(8):
                dv = dstblk[r, pl.ds(v * 16, 16)]
                sv = srcblk[r, pl.ds(v * 16, 16)]
                fl = plsc.load_gather(flagv, [dv])
                m = fl > 0
                o = jnp.minimum(off, _CCAP - 16)
                plsc.store_compressed(csb.at[pl.ds(o, 16)], sv, mask=m)
                plsc.store_compressed(cdb.at[pl.ds(o, 16)], dv, mask=m)
                cnt = plsc.all_reduce_population_count(m)
                off = off + cnt[0]
            return off

        lax.fori_loop(0, erpt, row, jnp.int32(0))
        pltpu.sync_copy(csb, csrc.at[wid])
        pltpu.sync_copy(cdb, cdst.at[wid])

    return pl.kernel(
        body,
        out_type=(jax.ShapeDtypeStruct((_NP, 128), jnp.float32),
                  jax.ShapeDtypeStruct((32, _CCAP), jnp.int32),
                  jax.ShapeDtypeStruct((32, _CCAP), jnp.int32)),
        mesh=_mesh(),
        compiler_params=pltpu.CompilerParams(needs_layout_passes=False),
        scratch_types=(
            pltpu.VMEM((1, 128), jnp.int32),
            pltpu.VMEM((128, 128), jnp.float32),
            pltpu.VMEM((_NP,), jnp.int32),
            pltpu.VMEM((erpt, 128), jnp.int32),
            pltpu.VMEM((erpt, 128), jnp.int32),
            pltpu.VMEM((_CCAP,), jnp.int32),
            pltpu.VMEM((_CCAP,), jnp.int32),
            pltpu.SemaphoreType.DMA,
        ))


@functools.lru_cache(maxsize=4)
def _make_segsum(split_by_core: bool, with_deg: bool, er: int):
    outs = [jax.ShapeDtypeStruct((2, _NP, 128), jnp.float32)]
    if with_deg:
        outs.append(jax.ShapeDtypeStruct((32, _NP), jnp.float32))
    scratch = [
        pltpu.VMEM((16, 128), jnp.int32),
        pltpu.VMEM((16, 128), jnp.int32),
        pltpu.VMEM((2 * 128, 128), jnp.float32),
        pltpu.VMEM_SHARED((_NP, 128), jnp.float32),
        pltpu.SemaphoreType.DMA,
        pltpu.SemaphoreType.DMA,
        pltpu.SemaphoreType.DMA,
        pltpu.SemaphoreType.DMA,
        pltpu.SemaphoreType.DMA,
        pltpu.SemaphoreType.DMA,
    ]
    if not split_by_core:
        scratch.append(pltpu.VMEM((16, 128), jnp.int32))
    if with_deg:
        scratch.append(pltpu.VMEM((_NP,), jnp.float32))

    def body(*refs):
        idxv = None
        if with_deg:
            (tab, src2d, dst2d, zrows, zvec,
             out, outdeg, srcv, dstv, rows, acc,
             sg0, sg1, ss0, ss1, sb0, sb1, degpart) = refs
        else:
            (tab, src2d, dst2d, zrows,
             out, srcv, dstv, rows, acc,
             sg0, sg1, ss0, ss1, sb0, sb1, idxv) = refs
        semg = (sg0, sg1)
        sems = (ss0, ss1)
        semb = (sb0, sb1)
        c = lax.axis_index("c")
        s = lax.axis_index("s")
        r0 = s * _ROWS_PER_TILE
        pltpu.sync_copy(zrows.at[pl.ds(r0, _ROWS_PER_TILE)],
                        acc.at[pl.ds(r0, _ROWS_PER_TILE)])
        if with_deg:
            pltpu.sync_copy(zvec, degpart)
        plsc.subcore_barrier()

        if split_by_core:
            wid = c * 16 + s
            nstep = er // 32
            rb0 = wid * nstep
        else:
            nstep = er // 16
            rb0 = s * nstep
        coff = c * _NP
        nsb = nstep // 8

        ones16 = jnp.full((16,), 1.0, jnp.float32)

        def sblock_descs(sb, sbp):
            return (
                pltpu.make_async_copy(src2d.at[pl.ds(rb0 + sb * 8, 8)],
                                      srcv.at[pl.ds(sbp * 8, 8)], semb[sbp]),
                pltpu.make_async_copy(dst2d.at[pl.ds(rb0 + sb * 8, 8)],
                                      dstv.at[pl.ds(sbp * 8, 8)], semb[sbp]),
            )

        def gather_desc(row, rp):
            idx = srcv if split_by_core else idxv
            return pltpu.make_async_copy(
                tab.at[idx.at[row]], rows.at[pl.ds(rp * 128, 128)],
                semg[rp])

        def scatter_desc(row, rp):
            return pltpu.make_async_copy(
                rows.at[pl.ds(rp * 128, 128)], acc.at[dstv.at[row]],
                sems[rp])

        for d in sblock_descs(0, 0):
            d.start()

        def pairblock(i, carry):
            for sbp in (0, 1):
                sb = 2 * i + sbp
                for s8 in range(8):
                    step = sb * 8 + s8
                    rp = s8 % 2
                    row = sbp * 8 + s8

                    @pl.when(step >= 1)
                    def _():
                        scatter_desc(row, 1 - rp).wait()

                    if s8 == 0:
                        @pl.when(sb + 1 < nsb)
                        def _():
                            for d in sblock_descs(sb + 1, 1 - sbp):
                                d.start()

                        for d in sblock_descs(sb, sbp):
                            d.wait()
                        if not split_by_core:
                            for rr in range(8):
                                for v in range(8):
                                    sv = srcv[sbp * 8 + rr,
                                              pl.ds(v * 16, 16)]
                                    idxv[sbp * 8 + rr,
                                         pl.ds(v * 16, 16)] = sv + coff
                        gather_desc(row, 0).start()
                        gather_desc(row + 1, 1).start()
                    elif s8 < 7:
                        gather_desc(row + 1, 1 - rp).start()

                    gather_desc(row, rp).wait()
                    pltpu.async_copy(rows.at[pl.ds(rp * 128, 128)],
                                     acc.at[dstv.at[row]], sems[rp],
                                     add=True)
                    if with_deg:
                        for v in range(8):
                            dv = dstv[row, pl.ds(v * 16, 16)]
                            plsc.addupdate_scatter(degpart, [dv], ones16)
            return carry

        lax.fori_loop(0, nsb // 2, pairblock, 0)
        scatter_desc(15, 1).wait()
        plsc.subcore_barrier()
        pltpu.sync_copy(acc.at[pl.ds(r0, _ROWS_PER_TILE)],
                        out.at[c, pl.ds(r0, _ROWS_PER_TILE)])
        if with_deg:
            pltpu.sync_copy(degpart, outdeg.at[c * 16 + s])

    out_type = tuple(outs) if len(outs) > 1 else outs[0]
    return pl.kernel(
        body, out_type=out_type, mesh=_mesh(),
        compiler_params=pltpu.CompilerParams(needs_layout_passes=False),
        scratch_types=tuple(scratch))


_BLK = 1024
_GRID = _NP // _BLK


def _vec_spec():
    return pl.BlockSpec((_BLK, 1), lambda i: (i, 0))


def _mat_spec():
    return pl.BlockSpec((_BLK, 128), lambda i: (i, 0))


def _deg_spec():
    return pl.BlockSpec((32, _BLK), lambda i: (0, i))


def _deg_of(dr):
    return jnp.maximum(jnp.sum(dr[...], axis=0), 1.0)[:, None]


def _tc_encode1(p0, p1, dg, w1):
    def body(p0r, p1r, dgr, w1r, outr):
        deg = _deg_of(dgr)
        agg = (p0r[...] + p1r[...]) / deg
        h = jnp.dot(agg, w1r[...], preferred_element_type=jnp.float32)
        h = jnp.maximum(h, 0.0)
        outr[0] = h[:, :128]
        outr[1] = h[:, 128:]

    return pl.pallas_call(
        body,
        grid=(_GRID,),
        in_specs=[_mat_spec(), _mat_spec(), _deg_spec(),
                  pl.BlockSpec((128, _H), lambda i: (0, 0))],
        out_specs=pl.BlockSpec((2, _BLK, 128), lambda i: (0, i, 0)),
        out_shape=jax.ShapeDtypeStruct((2, _NP, 128), jnp.float32),
    )(p0, p1, dg, w1)


def _tc_encode2(alo, ahi, dg, w2, we2d, ca, cb, rm, retok):
    def body(alor, ahir, dgr, w2r, wer, car, cbr, rmr, rtr, outr):
        deg = _deg_of(dgr)
        w2 = w2r[...]
        enc = (jnp.dot(alor[...] / deg, w2[:128],
                       preferred_element_type=jnp.float32) +
               jnp.dot(ahir[...] / deg, w2[128:],
                       preferred_element_type=jnp.float32))
        enc = jnp.maximum(enc, 0.0)
        rep = jnp.dot(enc, wer[...], preferred_element_type=jnp.float32)
        rep = car[...] * rep + cbr[...] + rmr[...] * rtr[...][0]
        outr[0] = rep[:, :128]
        outr[1] = rep[:, 128:]

    return pl.pallas_call(
        body,
        grid=(_GRID,),
        in_specs=[_mat_spec(), _mat_spec(), _deg_spec(),
                  pl.BlockSpec((_H, _H), lambda i: (0, 0)),
                  pl.BlockSpec((_H, _H), lambda i: (0, 0)),
                  _vec_spec(),
                  pl.BlockSpec((_BLK, _H), lambda i: (i, 0)),
                  _vec_spec(),
                  pl.BlockSpec((8, _H), lambda i: (0, 0))],
        out_specs=pl.BlockSpec((2, _BLK, 128), lambda i: (0, i, 0)),
        out_shape=jax.ShapeDtypeStruct((2, _NP, 128), jnp.float32),
    )(alo, ahi, dg, w2, we2d, ca, cb, rm, retok)


def _tc_decode_loss(alo, ahi, dg, xp, wd, wm):
    def body(alor, ahir, dgr, xr, wdr, wmr, outr):
        i = pl.program_id(0)
        deg = _deg_of(dgr)
        wd = wdr[...]
        y = (jnp.dot(alor[...] / deg, wd[:128],
                     preferred_element_type=jnp.float32) +
             jnp.dot(ahir[...] / deg, wd[128:],
                     preferred_element_type=jnp.float32))
        x = xr[...]
        xn = x / (jnp.sqrt(jnp.sum(x * x, axis=-1, keepdims=True)) + 1e-8)
        yn = y / (jnp.sqrt(jnp.sum(y * y, axis=-1, keepdims=True)) + 1e-8)
        cos = jnp.sum(xn * yn, axis=-1, keepdims=True)
        li = (1.0 - cos) ** 2 * wmr[...]
        part = jnp.sum(li) * (1.0 / 3000.0)

        @pl.when(i == 0)
        def _():
            outr[...] = jnp.zeros_like(outr)

        outr[...] += part

    return pl.pallas_call(
        body,
        grid=(_GRID,),
        in_specs=[_mat_spec(), _mat_spec(), _deg_spec(),
                  _mat_spec(),
                  pl.BlockSpec((_H, 128), lambda i: (0, 0)),
                  _vec_spec()],
        out_specs=pl.BlockSpec((8, 128), lambda i: (0, 0)),
        out_shape=jax.ShapeDtypeStruct((8, 128), jnp.float32),
    )(alo, ahi, dg, xp, wd, wm)


_CS = _mask_consts()


def kernel(x, edge_index, epoch, W1, W2, enc_mask_token, W_e2d,
           re_enc_mask_token, Wd):
    cs = _CS
    f32 = jnp.float32

    tab1 = jnp.concatenate(
        [x, enc_mask_token,
         jnp.zeros((_NP - _N - 1, _D), f32)], axis=0)
    xp = jnp.concatenate([x, jnp.zeros((_NP - _N, _D), f32)], axis=0)

    src = jnp.concatenate(
        [edge_index[0], jnp.zeros((_EP - _E,), jnp.int32)]).reshape(_ER, 128)
    pad_dst = _DUMMY + (jnp.arange(_EP - _E, dtype=jnp.int32) % 128)
    dst = jnp.concatenate(
        [edge_index[1], pad_dst]).reshape(_ER, 128)

    zrows = jnp.zeros((_NP, 128), f32)
    zvec = jnp.zeros((_NP,), f32)
    g2d = jnp.asarray(cs["g"]).reshape(_NP // 128, 128)
    flag = jnp.asarray(cs["flag"])
    fsrc = jnp.zeros((_CCAP,), jnp.int32)
    fdst = _DUMMY + (jnp.arange(_CCAP, dtype=jnp.int32) % 128)

    outx, csrc, cdst = _make_prelude()(tab1, g2d, src, dst, flag, fsrc, fdst)
    csrc = csrc.reshape(_CER, 128)
    cdst = cdst.reshape(_CER, 128)

    seg1 = _make_segsum(True, True, _ER)
    p, dg = seg1(outx, src, dst, zrows, zvec)

    h1 = _tc_encode1(p[0], p[1], dg, W1)

    seg2 = _make_segsum(False, False, _ER)
    a2 = seg2(h1.reshape(2 * _NP, 128), src, dst, zrows)

    retok = jnp.broadcast_to(re_enc_mask_token, (8, _H))
    rep = _tc_encode2(a2[0], a2[1], dg, W2, W_e2d,
                      jnp.asarray(cs["ca"]), jnp.asarray(cs["cb"]),
                      jnp.asarray(cs["rm"]), retok)

    seg3 = _make_segsum(False, False, _CER)
    a3 = seg3(rep.reshape(2 * _NP, 128), csrc, cdst, zrows)

    out = _tc_decode_loss(a3[0], a3[1], dg, xp, Wd,
                          jnp.asarray(cs["wm"]))
    return out[0, 0]


# speedup vs baseline: 3.9593x; 3.9569x over previous
import functools

import numpy as np
import jax
import jax.numpy as jnp
from jax import lax
from jax.experimental import pallas as pl
from jax.experimental.pallas import tpu as pltpu
from jax.experimental.pallas import tpu_sc as plsc

_N, _E, _D, _H = 10000, 320000, 128, 256
_NP = 10240
_EP = 327680
_ER = _EP // 128
_DUMMY = 10100
_ROWS_PER_TILE = _NP // 16
_CCAP = 4096
_CER = 32 * _CCAP // 128

_TIMESTEP, _START_T = 10000, 9000
_betas = np.linspace(1e-4, 0.02, _TIMESTEP, dtype=np.float64)
_ac = np.cumprod(1.0 - _betas)
_SQRT_AC = np.sqrt(_ac).astype(np.float32)
_SQRT_1MAC = np.sqrt(1.0 - _ac).astype(np.float32)


def _np(a):
    return np.asarray(jax.device_get(a))


@functools.lru_cache(maxsize=1)
def _mask_consts():
    with jax.default_device(jax.local_devices(backend="cpu")[0]):
        return _mask_consts_impl()


def _mask_consts_impl():
    mkey = jax.random.key(42)
    k1, k2, k3, k4, k5, k6 = jax.random.split(mkey, 6)
    n = _N
    num_mask = int(0.3 * n)
    num_noise = int(0.1 * num_mask)
    perm = _np(jax.random.permutation(k1, n))
    mask_nodes = perm[:num_mask]
    perm_mask = _np(jax.random.permutation(k2, num_mask))
    token_nodes = mask_nodes[perm_mask[: int(0.9 * num_mask)]]
    noise_nodes = mask_nodes[perm_mask[num_mask - num_noise:]]
    noise_chosen = _np(jax.random.permutation(k3, n))[:num_noise]
    t = _np(jax.random.randint(k4, (num_mask,), _START_T, _TIMESTEP))
    noise = _np(jax.random.normal(k5, (num_mask, _H), dtype=jnp.float32))
    perm_idx = _np(jax.random.permutation(k6, num_mask))
    remask_nodes = mask_nodes[perm_idx[: int(0.6 * num_mask)]]

    tf = t.astype(np.float32)
    a_c = _SQRT_AC[t] / tf
    b_c = _SQRT_1MAC[t] / tf

    g = np.arange(_NP, dtype=np.int32)
    g[token_nodes] = _N
    g[noise_nodes] = noise_chosen

    ca = np.ones((_NP, 1), np.float32)
    ca[mask_nodes, 0] = a_c
    ca[remask_nodes, 0] = 0.0
    cb = np.zeros((_NP, _H), np.float32)
    cb[mask_nodes] = b_c[:, None] * noise
    cb[remask_nodes] = 0.0
    rm = np.zeros((_NP, 1), np.float32)
    rm[remask_nodes, 0] = 1.0
    wm = np.zeros((_NP, 1), np.float32)
    wm[mask_nodes, 0] = 1.0
    flag = np.zeros((_NP,), np.int32)
    flag[mask_nodes] = 1
    return dict(g=g, ca=ca, cb=cb, rm=rm, wm=wm, flag=flag)


def _mesh():
    return plsc.VectorSubcoreMesh(
        core_axis_name="c", subcore_axis_name="s", num_cores=2,
        num_subcores=16)


@functools.lru_cache(maxsize=1)
def _make_prelude():
    nchunk = _NP // 128
    erpt = _ER // 32
    crpt = _CCAP // 128

    def body(tab, g2d, src2d, dst2d, flag_h, fsrc_h, fdst_h,
             outx, csrc, cdst,
             idxv, rowsb, flagv, srcblk, dstblk, csb, cdb, sem):
        wid = lax.axis_index("c") * 16 + lax.axis_index("s")
        for b in range(3):
            cid = wid + b * 32

            @pl.when(cid < nchunk)
            def _():
                pltpu.sync_copy(g2d.at[pl.ds(cid, 1)], idxv)
                pltpu.async_copy(tab.at[idxv.at[0]], rowsb, sem).wait()
                pltpu.sync_copy(rowsb, outx.at[pl.ds(cid * 128, 128)])

        pltpu.sync_copy(flag_h, flagv)
        pltpu.sync_copy(src2d.at[pl.ds(wid * erpt, erpt)], srcblk)
        pltpu.sync_copy(dst2d.at[pl.ds(wid * erpt, erpt)], dstblk)
        pltpu.sync_copy(fsrc_h, csb)
        pltpu.sync_copy(fdst_h, cdb)

        def row(r, off):
            for v in range(8):
                dv = dstblk[r, pl.ds(v * 16, 16)]
                sv = srcblk[r, pl.ds(v * 16, 16)]
                fl = plsc.load_gather(flagv, [dv])
                m = fl > 0
                o = jnp.minimum(off, _CCAP - 16)
                plsc.store_compressed(csb.at[pl.ds(o, 16)], sv, mask=m)
                plsc.store_compressed(cdb.at[pl.ds(o, 16)], dv, mask=m)
                cnt = plsc.all_reduce_population_count(m)
                off = off + cnt[0]
            return off

        lax.fori_loop(0, erpt, row, jnp.int32(0))
        pltpu.sync_copy(csb, csrc.at[wid])
        pltpu.sync_copy(cdb, cdst.at[wid])

    return pl.kernel(
        body,
        out_type=(jax.ShapeDtypeStruct((_NP, 128), jnp.float32),
                  jax.ShapeDtypeStruct((32, _CCAP), jnp.int32),
                  jax.ShapeDtypeStruct((32, _CCAP), jnp.int32)),
        mesh=_mesh(),
        compiler_params=pltpu.CompilerParams(needs_layout_passes=False),
        scratch_types=(
            pltpu.VMEM((1, 128), jnp.int32),
            pltpu.VMEM((128, 128), jnp.float32),
            pltpu.VMEM((_NP,), jnp.int32),
            pltpu.VMEM((erpt, 128), jnp.int32),
            pltpu.VMEM((erpt, 128), jnp.int32),
            pltpu.VMEM((_CCAP,), jnp.int32),
            pltpu.VMEM((_CCAP,), jnp.int32),
            pltpu.SemaphoreType.DMA,
        ))


@functools.lru_cache(maxsize=4)
def _make_segsum(split_by_core: bool, with_deg: bool, er: int):
    outs = [jax.ShapeDtypeStruct((2, _NP, 128), jnp.float32)]
    if with_deg:
        outs.append(jax.ShapeDtypeStruct((32, _NP), jnp.float32))
    scratch = [
        pltpu.VMEM((16, 128), jnp.int32),
        pltpu.VMEM((16, 128), jnp.int32),
        pltpu.VMEM((2 * 128, 128), jnp.float32),
        pltpu.VMEM_SHARED((_NP, 128), jnp.float32),
        pltpu.SemaphoreType.DMA,
        pltpu.SemaphoreType.DMA,
        pltpu.SemaphoreType.DMA,
        pltpu.SemaphoreType.DMA,
        pltpu.SemaphoreType.DMA,
        pltpu.SemaphoreType.DMA,
    ]
    if not split_by_core:
        scratch.append(pltpu.VMEM((16, 128), jnp.int32))
    if with_deg:
        scratch.append(pltpu.VMEM((_NP,), jnp.float32))

    def body(*refs):
        idxv = None
        if with_deg:
            (tab, src2d, dst2d, zrows, zvec,
             out, outdeg, srcv, dstv, rows, acc,
             sg0, sg1, ss0, ss1, sb0, sb1, degpart) = refs
        else:
            (tab, src2d, dst2d, zrows,
             out, srcv, dstv, rows, acc,
             sg0, sg1, ss0, ss1, sb0, sb1, idxv) = refs
        semg = (sg0, sg1)
        sems = (ss0, ss1)
        semb = (sb0, sb1)
        c = lax.axis_index("c")
        s = lax.axis_index("s")
        r0 = s * _ROWS_PER_TILE
        pltpu.sync_copy(zrows.at[pl.ds(r0, _ROWS_PER_TILE)],
                        acc.at[pl.ds(r0, _ROWS_PER_TILE)])
        if with_deg:
            pltpu.sync_copy(zvec, degpart)
        plsc.subcore_barrier()

        if split_by_core:
            wid = c * 16 + s
            nstep = er // 32
            rb0 = wid * nstep
        else:
            nstep = er // 16
            rb0 = s * nstep
        coff = c * _NP
        nsb = nstep // 8

        ones16 = jnp.full((16,), 1.0, jnp.float32)

        def sblock_descs(sb, sbp):
            return (
                pltpu.make_async_copy(src2d.at[pl.ds(rb0 + sb * 8, 8)],
                                      srcv.at[pl.ds(sbp * 8, 8)], semb[sbp]),
                pltpu.make_async_copy(dst2d.at[pl.ds(rb0 + sb * 8, 8)],
                                      dstv.at[pl.ds(sbp * 8, 8)], semb[sbp]),
            )

        def gather_desc(row, rp):
            idx = srcv if split_by_core else idxv
            return pltpu.make_async_copy(
                tab.at[idx.at[row]], rows.at[pl.ds(rp * 128, 128)],
                semg[rp])

        def scatter_desc(row, rp):
            return pltpu.make_async_copy(
                rows.at[pl.ds(rp * 128, 128)], acc.at[dstv.at[row]],
                sems[rp])

        for d in sblock_descs(0, 0):
            d.start()

        def pairblock(i, carry):
            for sbp in (0, 1):
                sb = 2 * i + sbp
                for s8 in range(8):
                    step = sb * 8 + s8
                    rp = s8 % 2
                    row = sbp * 8 + s8

                    @pl.when(step >= 1)
                    def _():
                        scatter_desc(row, 1 - rp).wait()

                    if s8 == 0:
                        @pl.when(sb + 1 < nsb)
                        def _():
                            for d in sblock_descs(sb + 1, 1 - sbp):
                                d.start()

                        for d in sblock_descs(sb, sbp):
                            d.wait()
                        if not split_by_core:
                            for rr in range(8):
                                for v in range(8):
                                    sv = srcv[sbp * 8 + rr,
                                              pl.ds(v * 16, 16)]
                                    idxv[sbp * 8 + rr,
                                         pl.ds(v * 16, 16)] = sv + coff
                        gather_desc(row, 0).start()
                        gather_desc(row + 1, 1).start()
                    elif s8 < 7:
                        gather_desc(row + 1, 1 - rp).start()

                    gather_desc(row, rp).wait()
                    pltpu.async_copy(rows.at[pl.ds(rp * 128, 128)],
                                     acc.at[dstv.at[row]], sems[rp],
                                     add=True)
                    if with_deg:
                        for v in range(8):
                            dv = dstv[row, pl.ds(v * 16, 16)]
                            plsc.addupdate_scatter(degpart, [dv], ones16)
            return carry

        lax.fori_loop(0, nsb // 2, pairblock, 0)
        scatter_desc(15, 1).wait()
        plsc.subcore_barrier()
        pltpu.sync_copy(acc.at[pl.ds(r0, _ROWS_PER_TILE)],
                        out.at[c, pl.ds(r0, _ROWS_PER_TILE)])
        if with_deg:
            pltpu.sync_copy(degpart, outdeg.at[c * 16 + s])

    out_type = tuple(outs) if len(outs) > 1 else outs[0]
    return pl.kernel(
        body, out_type=out_type, mesh=_mesh(),
        compiler_params=pltpu.CompilerParams(needs_layout_passes=False),
        scratch_types=tuple(scratch))


_BLK = 1024
_GRID = _NP // _BLK


def _vec_spec():
    return pl.BlockSpec((_BLK, 1), lambda i: (i, 0))


def _mat_spec():
    return pl.BlockSpec((_BLK, 128), lambda i: (i, 0))


def _deg_spec():
    return pl.BlockSpec((32, _BLK), lambda i: (0, i))


def _deg_of(dr):
    return jnp.maximum(jnp.sum(dr[...], axis=0), 1.0)[:, None]


def _tc_encode1(p0, p1, dg, w1):
    def body(p0r, p1r, dgr, w1r, outr):
        deg = _deg_of(dgr)
        agg = (p0r[...] + p1r[...]) / deg
        h = jnp.dot(agg, w1r[...], preferred_element_type=jnp.float32)
        h = jnp.maximum(h, 0.0)
        outr[0] = h[:, :128]
        outr[1] = h[:, 128:]

    return pl.pallas_call(
        body,
        grid=(_GRID,),
        in_specs=[_mat_spec(), _mat_spec(), _deg_spec(),
                  pl.BlockSpec((128, _H), lambda i: (0, 0))],
        out_specs=pl.BlockSpec((2, _BLK, 128), lambda i: (0, i, 0)),
        out_shape=jax.ShapeDtypeStruct((2, _NP, 128), jnp.float32),
    )(p0, p1, dg, w1)


def _tc_encode2(alo, ahi, dg, w2, we2d, ca, cb, rm, retok):
    def body(alor, ahir, dgr, w2r, wer, car, cbr, rmr, rtr, outr):
        deg = _deg_of(dgr)
        w2 = w2r[...]
        enc = (jnp.dot(alor[...] / deg, w2[:128],
                       preferred_element_type=jnp.float32) +
               jnp.dot(ahir[...] / deg, w2[128:],
                       preferred_element_type=jnp.float32))
        enc = jnp.maximum(enc, 0.0)
        rep = jnp.dot(enc, wer[...], preferred_element_type=jnp.float32)
        rep = car[...] * rep + cbr[...] + rmr[...] * rtr[...][0]
        outr[0] = rep[:, :128]
        outr[1] = rep[:, 128:]

    return pl.pallas_call(
        body,
        grid=(_GRID,),
        in_specs=[_mat_spec(), _mat_spec(), _deg_spec(),
                  pl.BlockSpec((_H, _H), lambda i: (0, 0)),
                  pl.BlockSpec((_H, _H), lambda i: (0, 0)),
                  _vec_spec(),
                  pl.BlockSpec((_BLK, _H), lambda i: (i, 0)),
                  _vec_spec(),
                  pl.BlockSpec((8, _H), lambda i: (0, 0))],
        out_specs=pl.BlockSpec((2, _BLK, 128), lambda i: (0, i, 0)),
        out_shape=jax.ShapeDtypeStruct((2, _NP, 128), jnp.float32),
    )(alo, ahi, dg, w2, we2d, ca, cb, rm, retok)


def _tc_decode_loss(alo, ahi, dg, xp, wd, wm):
    def body(alor, ahir, dgr, xr, wdr, wmr, outr):
        i = pl.program_id(0)
        deg = _deg_of(dgr)
        wd = wdr[...]
        y = (jnp.dot(alor[...] / deg, wd[:128],
                     preferred_element_type=jnp.float32) +
             jnp.dot(ahir[...] / deg, wd[128:],
                     preferred_element_type=jnp.float32))
        x = xr[...]
        xn = x / (jnp.sqrt(jnp.sum(x * x, axis=-1, keepdims=True)) + 1e-8)
        yn = y / (jnp.sqrt(jnp.sum(y * y, axis=-1, keepdims=True)) + 1e-8)
        cos = jnp.sum(xn * yn, axis=-1, keepdims=True)
        li = (1.0 - cos) ** 2 * wmr[...]
        part = jnp.sum(li) * (1.0 / 3000.0)

        @pl.when(i == 0)
        def _():
            outr[...] = jnp.zeros_like(outr)

        outr[...] += part

    return pl.pallas_call(
        body,
        grid=(_GRID,),
        in_specs=[_mat_spec(), _mat_spec(), _deg_spec(),
                  _mat_spec(),
                  pl.BlockSpec((_H, 128), lambda i: (0, 0)),
                  _vec_spec()],
        out_specs=pl.BlockSpec((8, 128), lambda i: (0, 0)),
        out_shape=jax.ShapeDtypeStruct((8, 128), jnp.float32),
    )(alo, ahi, dg, xp, wd, wm)


_CS = _mask_consts()


def kernel(x, edge_index, epoch, W1, W2, enc_mask_token, W_e2d,
           re_enc_mask_token, Wd):
    cs = _CS
    f32 = jnp.float32

    tab1 = jnp.concatenate(
        [x, enc_mask_token,
         jnp.zeros((_NP - _N - 1, _D), f32)], axis=0)
    xp = jnp.concatenate([x, jnp.zeros((_NP - _N, _D), f32)], axis=0)

    pad_spread = _DUMMY + (jnp.arange(_EP - _E, dtype=jnp.int32) % 128)
    src = jnp.concatenate(
        [edge_index[0], pad_spread]).reshape(_ER, 128)
    pad_dst = pad_spread
    dst = jnp.concatenate(
        [edge_index[1], pad_dst]).reshape(_ER, 128)

    zrows = jnp.zeros((_NP, 128), f32)
    zvec = jnp.zeros((_NP,), f32)
    g2d = jnp.asarray(cs["g"]).reshape(_NP // 128, 128)
    flag = jnp.asarray(cs["flag"])
    fsrc = _DUMMY + (jnp.arange(_CCAP, dtype=jnp.int32) % 128)
    fdst = _DUMMY + (jnp.arange(_CCAP, dtype=jnp.int32) % 128)

    outx, csrc, cdst = _make_prelude()(tab1, g2d, src, dst, flag, fsrc, fdst)
    csrc = csrc.reshape(_CER, 128)
    cdst = cdst.reshape(_CER, 128)

    seg1 = _make_segsum(True, True, _ER)
    p, dg = seg1(outx, src, dst, zrows, zvec)

    h1 = _tc_encode1(p[0], p[1], dg, W1)

    seg2 = _make_segsum(False, False, _ER)
    a2 = seg2(h1.reshape(2 * _NP, 128), src, dst, zrows)

    retok = jnp.broadcast_to(re_enc_mask_token, (8, _H))
    rep = _tc_encode2(a2[0], a2[1], dg, W2, W_e2d,
                      jnp.asarray(cs["ca"]), jnp.asarray(cs["cb"]),
                      jnp.asarray(cs["rm"]), retok)

    seg3 = _make_segsum(False, False, _CER)
    a3 = seg3(rep.reshape(2 * _NP, 128), csrc, cdst, zrows)

    out = _tc_decode_loss(a3[0], a3[1], dg, xp, Wd,
                          jnp.asarray(cs["wm"]))
    return out[0, 0]


# speedup vs baseline: 3.9885x; 1.0074x over previous
import functools

import numpy as np
import jax
import jax.numpy as jnp
from jax import lax
from jax.experimental import pallas as pl
from jax.experimental.pallas import tpu as pltpu
from jax.experimental.pallas import tpu_sc as plsc

_N, _E, _D, _H = 10000, 320000, 128, 256
_NP = 10240
_EP = 327680
_ER = _EP // 128
_DUMMY = 10100
_ROWS_PER_TILE = _NP // 16
_CCAP = 4096
_CER = 32 * _CCAP // 128

_TIMESTEP, _START_T = 10000, 9000
_betas = np.linspace(1e-4, 0.02, _TIMESTEP, dtype=np.float64)
_ac = np.cumprod(1.0 - _betas)
_SQRT_AC = np.sqrt(_ac).astype(np.float32)
_SQRT_1MAC = np.sqrt(1.0 - _ac).astype(np.float32)


def _np(a):
    return np.asarray(jax.device_get(a))


@functools.lru_cache(maxsize=1)
def _mask_consts():
    with jax.default_device(jax.local_devices(backend="cpu")[0]):
        return _mask_consts_impl()


def _mask_consts_impl():
    mkey = jax.random.key(42)
    k1, k2, k3, k4, k5, k6 = jax.random.split(mkey, 6)
    n = _N
    num_mask = int(0.3 * n)
    num_noise = int(0.1 * num_mask)
    perm = _np(jax.random.permutation(k1, n))
    mask_nodes = perm[:num_mask]
    perm_mask = _np(jax.random.permutation(k2, num_mask))
    token_nodes = mask_nodes[perm_mask[: int(0.9 * num_mask)]]
    noise_nodes = mask_nodes[perm_mask[num_mask - num_noise:]]
    noise_chosen = _np(jax.random.permutation(k3, n))[:num_noise]
    t = _np(jax.random.randint(k4, (num_mask,), _START_T, _TIMESTEP))
    noise = _np(jax.random.normal(k5, (num_mask, _H), dtype=jnp.float32))
    perm_idx = _np(jax.random.permutation(k6, num_mask))
    remask_nodes = mask_nodes[perm_idx[: int(0.6 * num_mask)]]

    tf = t.astype(np.float32)
    a_c = _SQRT_AC[t] / tf
    b_c = _SQRT_1MAC[t] / tf

    g = np.arange(_NP, dtype=np.int32)
    g[token_nodes] = _N
    g[noise_nodes] = noise_chosen

    ca = np.ones((_NP, 1), np.float32)
    ca[mask_nodes, 0] = a_c
    ca[remask_nodes, 0] = 0.0
    cb = np.zeros((_NP, _H), np.float32)
    cb[mask_nodes] = b_c[:, None] * noise
    cb[remask_nodes] = 0.0
    rm = np.zeros((_NP, 1), np.float32)
    rm[remask_nodes, 0] = 1.0
    wm = np.zeros((_NP, 1), np.float32)
    wm[mask_nodes, 0] = 1.0
    flag = np.zeros((_NP,), np.int32)
    flag[mask_nodes] = 1
    return dict(g=g, ca=ca, cb=cb, rm=rm, wm=wm, flag=flag)


def _mesh():
    return plsc.VectorSubcoreMesh(
        core_axis_name="c", subcore_axis_name="s", num_cores=2,
        num_subcores=16)


@functools.lru_cache(maxsize=1)
def _make_prelude():
    nchunk = _NP // 128
    erpt = _ER // 32
    crpt = _CCAP // 128

    def body(tab, g2d, src2d, dst2d, flag_h, fsrc_h, fdst_h,
             outx, csrc, cdst,
             idxv, rowsb, flagv, srcblk, dstblk, csb, cdb, sem):
        wid = lax.axis_index("c") * 16 + lax.axis_index("s")
        descs = []
        for b in range(3):
            cid = wid + b * 32

            @pl.when(cid < nchunk)
            def _():
                pltpu.sync_copy(g2d.at[pl.ds(cid, 1)],
                                idxv.at[pl.ds(b, 1)])
                pltpu.make_async_copy(
                    tab.at[idxv.at[b]],
                    rowsb.at[pl.ds(b * 128, 128)], sem).start()
        pltpu.sync_copy(flag_h, flagv)
        pltpu.sync_copy(src2d.at[pl.ds(wid * erpt, erpt)], srcblk)
        pltpu.sync_copy(dst2d.at[pl.ds(wid * erpt, erpt)], dstblk)
        pltpu.sync_copy(fsrc_h, csb)
        pltpu.sync_copy(fdst_h, cdb)

        def row(r, off):
            for v in range(8):
                dv = dstblk[r, pl.ds(v * 16, 16)]
                sv = srcblk[r, pl.ds(v * 16, 16)]
                fl = plsc.load_gather(flagv, [dv])
                m = fl > 0
                o = jnp.minimum(off, _CCAP - 16)
                plsc.store_compressed(csb.at[pl.ds(o, 16)], sv, mask=m)
                plsc.store_compressed(cdb.at[pl.ds(o, 16)], dv, mask=m)
                cnt = plsc.all_reduce_population_count(m)
                off = off + cnt[0]
            return off

        lax.fori_loop(0, erpt, row, jnp.int32(0))
        pltpu.sync_copy(csb, csrc.at[wid])
        pltpu.sync_copy(cdb, cdst.at[wid])

        for b in range(3):
            cid = wid + b * 32

            @pl.when(cid < nchunk)
            def _():
                pltpu.make_async_copy(
                    tab.at[idxv.at[b]],
                    rowsb.at[pl.ds(b * 128, 128)], sem).wait()
                pltpu.sync_copy(rowsb.at[pl.ds(b * 128, 128)],
                                outx.at[pl.ds(cid * 128, 128)])

    return pl.kernel(
        body,
        out_type=(jax.ShapeDtypeStruct((_NP, 128), jnp.float32),
                  jax.ShapeDtypeStruct((32, _CCAP), jnp.int32),
                  jax.ShapeDtypeStruct((32, _CCAP), jnp.int32)),
        mesh=_mesh(),
        compiler_params=pltpu.CompilerParams(needs_layout_passes=False),
        scratch_types=(
            pltpu.VMEM((3, 128), jnp.int32),
            pltpu.VMEM((3 * 128, 128), jnp.float32),
            pltpu.VMEM((_NP,), jnp.int32),
            pltpu.VMEM((erpt, 128), jnp.int32),
            pltpu.VMEM((erpt, 128), jnp.int32),
            pltpu.VMEM((_CCAP,), jnp.int32),
            pltpu.VMEM((_CCAP,), jnp.int32),
            pltpu.SemaphoreType.DMA,
        ))


@functools.lru_cache(maxsize=4)
def _make_segsum(split_by_core: bool, with_deg: bool, er: int):
    outs = [jax.ShapeDtypeStruct((2, _NP, 128), jnp.float32)]
    if with_deg:
        outs.append(jax.ShapeDtypeStruct((32, _NP), jnp.float32))
    scratch = [
        pltpu.VMEM((16, 128), jnp.int32),
        pltpu.VMEM((16, 128), jnp.int32),
        pltpu.VMEM((2 * 128, 128), jnp.float32),
        pltpu.VMEM_SHARED((_NP, 128), jnp.float32),
        pltpu.SemaphoreType.DMA,
        pltpu.SemaphoreType.DMA,
        pltpu.SemaphoreType.DMA,
        pltpu.SemaphoreType.DMA,
        pltpu.SemaphoreType.DMA,
        pltpu.SemaphoreType.DMA,
    ]
    if not split_by_core:
        scratch.append(pltpu.VMEM((16, 128), jnp.int32))
    if with_deg:
        scratch.append(pltpu.VMEM((_NP,), jnp.float32))

    def body(*refs):
        idxv = None
        if with_deg:
            (tab, src2d, dst2d, zrows, zvec,
             out, outdeg, srcv, dstv, rows, acc,
             sg0, sg1, ss0, ss1, sb0, sb1, degpart) = refs
        else:
            (tab, src2d, dst2d, zrows,
             out, srcv, dstv, rows, acc,
             sg0, sg1, ss0, ss1, sb0, sb1, idxv) = refs
        semg = (sg0, sg1)
        sems = (ss0, ss1)
        semb = (sb0, sb1)
        c = lax.axis_index("c")
        s = lax.axis_index("s")
        r0 = s * _ROWS_PER_TILE
        pltpu.sync_copy(zrows.at[pl.ds(r0, _ROWS_PER_TILE)],
                        acc.at[pl.ds(r0, _ROWS_PER_TILE)])
        if with_deg:
            pltpu.sync_copy(zvec, degpart)
        plsc.subcore_barrier()

        if split_by_core:
            wid = c * 16 + s
            nstep = er // 32
            rb0 = wid * nstep
        else:
            nstep = er // 16
            rb0 = s * nstep
        coff = c * _NP
        nsb = nstep // 8

        ones16 = jnp.full((16,), 1.0, jnp.float32)

        def sblock_descs(sb, sbp):
            return (
                pltpu.make_async_copy(src2d.at[pl.ds(rb0 + sb * 8, 8)],
                                      srcv.at[pl.ds(sbp * 8, 8)], semb[sbp]),
                pltpu.make_async_copy(dst2d.at[pl.ds(rb0 + sb * 8, 8)],
                                      dstv.at[pl.ds(sbp * 8, 8)], semb[sbp]),
            )

        def gather_desc(row, rp):
            idx = srcv if split_by_core else idxv
            return pltpu.make_async_copy(
                tab.at[idx.at[row]], rows.at[pl.ds(rp * 128, 128)],
                semg[rp])

        def scatter_desc(row, rp):
            return pltpu.make_async_copy(
                rows.at[pl.ds(rp * 128, 128)], acc.at[dstv.at[row]],
                sems[rp])

        for d in sblock_descs(0, 0):
            d.start()

        def pairblock(i, carry):
            for sbp in (0, 1):
                sb = 2 * i + sbp
                for s8 in range(8):
                    step = sb * 8 + s8
                    rp = s8 % 2
                    row = sbp * 8 + s8

                    @pl.when(step >= 1)
                    def _():
                        scatter_desc(row, 1 - rp).wait()

                    if s8 == 0:
                        @pl.when(sb + 1 < nsb)
                        def _():
                            for d in sblock_descs(sb + 1, 1 - sbp):
                                d.start()

                        for d in sblock_descs(sb, sbp):
                            d.wait()
                        if not split_by_core:
                            for rr in range(8):
                                for v in range(8):
                                    sv = srcv[sbp * 8 + rr,
                                              pl.ds(v * 16, 16)]
                                    idxv[sbp * 8 + rr,
                                         pl.ds(v * 16, 16)] = sv + coff
                        gather_desc(row, 0).start()
                        gather_desc(row + 1, 1).start()
                    elif s8 < 7:
                        gather_desc(row + 1, 1 - rp).start()

                    gather_desc(row, rp).wait()
                    pltpu.async_copy(rows.at[pl.ds(rp * 128, 128)],
                                     acc.at[dstv.at[row]], sems[rp],
                                     add=True)
                    if with_deg:
                        for v in range(8):
                            dv = dstv[row, pl.ds(v * 16, 16)]
                            plsc.addupdate_scatter(degpart, [dv], ones16)
            return carry

        lax.fori_loop(0, nsb // 2, pairblock, 0)
        scatter_desc(15, 1).wait()
        plsc.subcore_barrier()
        pltpu.sync_copy(acc.at[pl.ds(r0, _ROWS_PER_TILE)],
                        out.at[c, pl.ds(r0, _ROWS_PER_TILE)])
        if with_deg:
            pltpu.sync_copy(degpart, outdeg.at[c * 16 + s])

    out_type = tuple(outs) if len(outs) > 1 else outs[0]
    return pl.kernel(
        body, out_type=out_type, mesh=_mesh(),
        compiler_params=pltpu.CompilerParams(needs_layout_passes=False),
        scratch_types=tuple(scratch))


_BLK = 1024
_GRID = _NP // _BLK


def _vec_spec():
    return pl.BlockSpec((_BLK, 1), lambda i: (i, 0))


def _mat_spec():
    return pl.BlockSpec((_BLK, 128), lambda i: (i, 0))


def _deg_spec():
    return pl.BlockSpec((32, _BLK), lambda i: (0, i))


def _deg_of(dr):
    return jnp.maximum(jnp.sum(dr[...], axis=0), 1.0)[:, None]


def _tc_encode1(p0, p1, dg, w1):
    def body(p0r, p1r, dgr, w1r, outr):
        deg = _deg_of(dgr)
        agg = (p0r[...] + p1r[...]) / deg
        h = jnp.dot(agg, w1r[...], preferred_element_type=jnp.float32)
        h = jnp.maximum(h, 0.0)
        outr[0] = h[:, :128]
        outr[1] = h[:, 128:]

    return pl.pallas_call(
        body,
        grid=(_GRID,),
        in_specs=[_mat_spec(), _mat_spec(), _deg_spec(),
                  pl.BlockSpec((128, _H), lambda i: (0, 0))],
        out_specs=pl.BlockSpec((2, _BLK, 128), lambda i: (0, i, 0)),
        out_shape=jax.ShapeDtypeStruct((2, _NP, 128), jnp.float32),
    )(p0, p1, dg, w1)


def _tc_encode2(alo, ahi, dg, w2, we2d, ca, cb, rm, retok):
    def body(alor, ahir, dgr, w2r, wer, car, cbr, rmr, rtr, outr):
        deg = _deg_of(dgr)
        w2 = w2r[...]
        enc = (jnp.dot(alor[...] / deg, w2[:128],
                       preferred_element_type=jnp.float32) +
               jnp.dot(ahir[...] / deg, w2[128:],
                       preferred_element_type=jnp.float32))
        enc = jnp.maximum(enc, 0.0)
        rep = jnp.dot(enc, wer[...], preferred_element_type=jnp.float32)
        rep = car[...] * rep + cbr[...] + rmr[...] * rtr[...][0]
        outr[0] = rep[:, :128]
        outr[1] = rep[:, 128:]

    return pl.pallas_call(
        body,
        grid=(_GRID,),
        in_specs=[_mat_spec(), _mat_spec(), _deg_spec(),
                  pl.BlockSpec((_H, _H), lambda i: (0, 0)),
                  pl.BlockSpec((_H, _H), lambda i: (0, 0)),
                  _vec_spec(),
                  pl.BlockSpec((_BLK, _H), lambda i: (i, 0)),
                  _vec_spec(),
                  pl.BlockSpec((8, _H), lambda i: (0, 0))],
        out_specs=pl.BlockSpec((2, _BLK, 128), lambda i: (0, i, 0)),
        out_shape=jax.ShapeDtypeStruct((2, _NP, 128), jnp.float32),
    )(alo, ahi, dg, w2, we2d, ca, cb, rm, retok)


def _tc_decode_loss(alo, ahi, dg, xp, wd, wm):
    def body(alor, ahir, dgr, xr, wdr, wmr, outr):
        i = pl.program_id(0)
        deg = _deg_of(dgr)
        wd = wdr[...]
        y = (jnp.dot(alor[...] / deg, wd[:128],
                     preferred_element_type=jnp.float32) +
             jnp.dot(ahir[...] / deg, wd[128:],
                     preferred_element_type=jnp.float32))
        x = xr[...]
        xn = x / (jnp.sqrt(jnp.sum(x * x, axis=-1, keepdims=True)) + 1e-8)
        yn = y / (jnp.sqrt(jnp.sum(y * y, axis=-1, keepdims=True)) + 1e-8)
        cos = jnp.sum(xn * yn, axis=-1, keepdims=True)
        li = (1.0 - cos) ** 2 * wmr[...]
        part = jnp.sum(li) * (1.0 / 3000.0)

        @pl.when(i == 0)
        def _():
            outr[...] = jnp.zeros_like(outr)

        outr[...] += part

    return pl.pallas_call(
        body,
        grid=(_GRID,),
        in_specs=[_mat_spec(), _mat_spec(), _deg_spec(),
                  _mat_spec(),
                  pl.BlockSpec((_H, 128), lambda i: (0, 0)),
                  _vec_spec()],
        out_specs=pl.BlockSpec((8, 128), lambda i: (0, 0)),
        out_shape=jax.ShapeDtypeStruct((8, 128), jnp.float32),
    )(alo, ahi, dg, xp, wd, wm)


_CS = _mask_consts()


def kernel(x, edge_index, epoch, W1, W2, enc_mask_token, W_e2d,
           re_enc_mask_token, Wd):
    cs = _CS
    f32 = jnp.float32

    tab1 = jnp.concatenate(
        [x, enc_mask_token,
         jnp.zeros((_NP - _N - 1, _D), f32)], axis=0)
    xp = jnp.concatenate([x, jnp.zeros((_NP - _N, _D), f32)], axis=0)

    pad_spread = _DUMMY + (jnp.arange(_EP - _E, dtype=jnp.int32) % 128)
    src = jnp.concatenate(
        [edge_index[0], pad_spread]).reshape(_ER, 128)
    pad_dst = pad_spread
    dst = jnp.concatenate(
        [edge_index[1], pad_dst]).reshape(_ER, 128)

    zrows = jnp.zeros((_NP, 128), f32)
    zvec = jnp.zeros((_NP,), f32)
    g2d = jnp.asarray(cs["g"]).reshape(_NP // 128, 128)
    flag = jnp.asarray(cs["flag"])
    fsrc = _DUMMY + (jnp.arange(_CCAP, dtype=jnp.int32) % 128)
    fdst = _DUMMY + (jnp.arange(_CCAP, dtype=jnp.int32) % 128)

    outx, csrc, cdst = _make_prelude()(tab1, g2d, src, dst, flag, fsrc, fdst)
    csrc = csrc.reshape(_CER, 128)
    cdst = cdst.reshape(_CER, 128)

    seg1 = _make_segsum(True, True, _ER)
    p, dg = seg1(outx, src, dst, zrows, zvec)

    h1 = _tc_encode1(p[0], p[1], dg, W1)

    seg2 = _make_segsum(False, False, _ER)
    a2 = seg2(h1.reshape(2 * _NP, 128), src, dst, zrows)

    retok = jnp.broadcast_to(re_enc_mask_token, (8, _H))
    rep = _tc_encode2(a2[0], a2[1], dg, W2, W_e2d,
                      jnp.asarray(cs["ca"]), jnp.asarray(cs["cb"]),
                      jnp.asarray(cs["rm"]), retok)

    seg3 = _make_segsum(False, False, _CER)
    a3 = seg3(rep.reshape(2 * _NP, 128), csrc, cdst, zrows)

    out = _tc_decode_loss(a3[0], a3[1], dg, xp, Wd,
                          jnp.asarray(cs["wm"]))
    return out[0, 0]
